# Initial kernel scaffold; baseline (speedup 1.0000x reference)
import jax
import jax.numpy as jnp
from jax.experimental import pallas as pl


def kernel(x, W_router, W_experts, b_experts):
    raise NotImplementedError("write your pallas kernel here")


# speedup vs baseline: 1.3469x; 1.3469x over previous
import functools

import jax
import jax.numpy as jnp
from jax import lax
from jax.experimental import pallas as pl
from jax.experimental.pallas import tpu as pltpu
from jax.experimental.pallas import tpu_sc as plsc

E = 8
K = 2
D = 2048
N = 8192
M = N * K

RBLK = 512
NB = N // RBLK

TM = 512
NT = M // TM
V = NT + E - 1

NC = 2
NS = 16
NW = NC * NS
TPW = N // NW
RSUB = 32
J = TPW // RSUB
CS = 16


def _router_body(x_ref, wr_ref, meta_ref, cnt_ref, carry_ref):
    b = pl.program_id(0)

    @pl.when(b == 0)
    def _():
        carry_ref[...] = jnp.zeros_like(carry_ref)

    xb = x_ref[...]
    logits = jnp.dot(xb, wr_ref[...], preferred_element_type=jnp.float32)
    lane = lax.broadcasted_iota(jnp.int32, (RBLK, 128), 1)
    valid = lane < E
    neg = jnp.float32(-1e30)
    lm = jnp.where(valid, logits, neg)
    m0 = jnp.max(lm, axis=1, keepdims=True)
    is0 = (lm == m0) & valid
    e0 = jnp.min(jnp.where(is0, lane, 127), axis=1, keepdims=True)
    sel0 = lane == e0
    lm1 = jnp.where(sel0, neg, lm)
    m1 = jnp.max(lm1, axis=1, keepdims=True)
    is1 = (lm1 == m1) & valid
    e1 = jnp.min(jnp.where(is1, lane, 127), axis=1, keepdims=True)
    sel1 = lane == e1
    t = jnp.exp(m1 - m0)
    w0 = 1.0 / (1.0 + t)
    w1 = t / (1.0 + t)
    oh = sel0.astype(jnp.float32) + sel1.astype(jnp.float32)
    ri = lax.broadcasted_iota(jnp.int32, (RBLK, RBLK), 0)
    ci = lax.broadcasted_iota(jnp.int32, (RBLK, RBLK), 1)
    tri = (ri > ci).astype(jnp.float32)
    excl = jnp.dot(tri, oh, preferred_element_type=jnp.float32)
    carry = carry_ref[0:1, :]
    rank = excl + carry
    rank0 = jnp.sum(jnp.where(sel0, rank, 0.0), axis=1, keepdims=True)
    rank1 = jnp.sum(jnp.where(sel1, rank, 0.0), axis=1, keepdims=True)
    tot = carry + jnp.sum(oh, axis=0, keepdims=True)
    carry_ref[...] = jnp.broadcast_to(tot, (8, 128))
    cnt_ref[...] = jnp.broadcast_to(tot, (8, 128)).reshape(1, 8, 128)
    meta_ref[...] = (
        jnp.where(lane == 0, e0.astype(jnp.float32), 0.0)
        + jnp.where(lane == 1, e1.astype(jnp.float32), 0.0)
        + jnp.where(lane == 2, w0, 0.0)
        + jnp.where(lane == 3, w1, 0.0)
        + jnp.where(lane == 4, rank0, 0.0)
        + jnp.where(lane == 5, rank1, 0.0)
        + jnp.where((lane >= 64) & (lane < 80), w0, 0.0)
        + jnp.where((lane >= 80) & (lane < 96), w1, 0.0)
    )


def _router(x, wr_pad, interpret=False):
    return pl.pallas_call(
        _router_body,
        grid=(NB,),
        in_specs=[
            pl.BlockSpec((RBLK, D), lambda b: (b, 0)),
            pl.BlockSpec((D, 128), lambda b: (0, 0)),
        ],
        out_specs=[
            pl.BlockSpec((RBLK, 128), lambda b: (b, 0)),
            pl.BlockSpec((1, 8, 128), lambda b: (b, 0, 0)),
        ],
        out_shape=[
            jax.ShapeDtypeStruct((N, 128), jnp.float32),
            jax.ShapeDtypeStruct((NB, 8, 128), jnp.float32),
        ],
        scratch_shapes=[pltpu.VMEM((8, 128), jnp.float32)],
        interpret=interpret,
    )(x, wr_pad)


def _gmm_body(vt_ref, ve_ref, vlo_ref, vhi_ref, vfv_ref, xs_ref, w_ref, b_ref,
              out_ref):
    v = pl.program_id(0)
    tile = vt_ref[v]
    lo = vlo_ref[v]
    hi = vhi_ref[v]
    fv = vfv_ref[v]

    @pl.when(hi > lo)
    def _():
        rows = lax.broadcasted_iota(jnp.int32, (TM, 1), 0) + tile * TM
        m = (rows >= lo) & (rows < hi)
        acc = jnp.dot(xs_ref[...], w_ref[0], preferred_element_type=jnp.float32)
        acc = jnp.where(m, acc + b_ref[0], 0.0)
        prev = out_ref[...]
        out_ref[...] = jnp.where(fv == 1, acc, prev + acc)


def _gmm(vt, ve, vlo, vhi, vfv, xs, w_experts, b3, interpret=False):
    spec = pltpu.PrefetchScalarGridSpec(
        num_scalar_prefetch=5,
        grid=(V,),
        in_specs=[
            pl.BlockSpec((TM, D), lambda v, vt, ve, vlo, vhi, vfv: (vt[v], 0)),
            pl.BlockSpec((1, D, D),
                         lambda v, vt, ve, vlo, vhi, vfv: (ve[v], 0, 0)),
            pl.BlockSpec((1, 1, D),
                         lambda v, vt, ve, vlo, vhi, vfv: (ve[v], 0, 0)),
        ],
        out_specs=pl.BlockSpec((TM, D),
                               lambda v, vt, ve, vlo, vhi, vfv: (vt[v], 0)),
    )
    return pl.pallas_call(
        _gmm_body,
        grid_spec=spec,
        out_shape=jax.ShapeDtypeStruct((M, D), jnp.float32),
        compiler_params=pltpu.CompilerParams(
            vmem_limit_bytes=100 * 1024 * 1024),
        interpret=interpret,
    )(vt, ve, vlo, vhi, vfv, xs, w_experts, b3)


def _dispatch(x, pos3d):
    mesh = plsc.VectorSubcoreMesh(core_axis_name="c", subcore_axis_name="s",
                                  num_cores=NC, num_subcores=NS)

    @functools.partial(
        pl.kernel,
        out_type=jax.ShapeDtypeStruct((M, D), jnp.float32),
        mesh=mesh,
        scratch_types=[
            pltpu.VMEM((2 * J, RSUB), jnp.int32),
            pltpu.VMEM((RSUB, D), jnp.float32),
            pltpu.SemaphoreType.DMA,
            pltpu.SemaphoreType.DMA,
        ],
    )
    def k(x_hbm, pos_hbm, xs_hbm, idx_v, rows_v, s0, s1):
        wid = lax.axis_index("s") * NC + lax.axis_index("c")
        base = wid * TPW
        pltpu.sync_copy(pos_hbm.at[wid], idx_v)

        def body(j, carry):
            pltpu.sync_copy(x_hbm.at[pl.ds(base + j * RSUB, RSUB)], rows_v)
            c0 = pltpu.async_copy(rows_v, xs_hbm.at[idx_v.at[j]], s0)
            c1 = pltpu.async_copy(rows_v, xs_hbm.at[idx_v.at[J + j]], s1)
            c0.wait()
            c1.wait()
            return carry

        lax.fori_loop(0, J, body, 0)

    return k(x, pos3d)


def _combine(ys, pos3d, w0e, w1e):
    mesh = plsc.VectorSubcoreMesh(core_axis_name="c", subcore_axis_name="s",
                                  num_cores=NC, num_subcores=NS)

    @functools.partial(
        pl.kernel,
        out_type=jax.ShapeDtypeStruct((N, D), jnp.float32),
        mesh=mesh,
        scratch_types=[
            pltpu.VMEM((2 * J, RSUB), jnp.int32),
            pltpu.VMEM((CS, D), jnp.float32),
            pltpu.VMEM((CS, D), jnp.float32),
            pltpu.VMEM((CS, D), jnp.float32),
            pltpu.VMEM((CS, 16), jnp.float32),
            pltpu.VMEM((CS, 16), jnp.float32),
            pltpu.SemaphoreType.DMA,
            pltpu.SemaphoreType.DMA,
        ],
    )
    def k(ys_hbm, pos_hbm, w0_hbm, w1_hbm, out_hbm, idx_v, r0_v, r1_v, o_v,
          w0_v, w1_v, s0, s1):
        wid = lax.axis_index("s") * NC + lax.axis_index("c")
        base = wid * TPW
        pltpu.sync_copy(pos_hbm.at[wid], idx_v)

        def body(h, carry):
            j = h // 2
            half = (h % 2) * CS
            c0 = pltpu.async_copy(ys_hbm.at[idx_v.at[j, pl.ds(half, CS)]],
                                  r0_v, s0)
            c1 = pltpu.async_copy(ys_hbm.at[idx_v.at[J + j, pl.ds(half, CS)]],
                                  r1_v, s1)
            pltpu.sync_copy(w0_hbm.at[pl.ds(base + h * CS, CS)], w0_v)
            pltpu.sync_copy(w1_hbm.at[pl.ds(base + h * CS, CS)], w1_v)
            c0.wait()
            c1.wait()

            def rbody(r, rc):
                w0s = w0_v[r, :]
                w1s = w1_v[r, :]

                def cbody(c, cc):
                    for u in range(8):
                        sl = pl.ds((c * 8 + u) * 16, 16)
                        o_v[r, sl] = w0s * r0_v[r, sl] + w1s * r1_v[r, sl]
                    return cc

                lax.fori_loop(0, D // 128, cbody, 0)
                return rc

            lax.fori_loop(0, CS, rbody, 0)
            pltpu.sync_copy(o_v, out_hbm.at[pl.ds(base + h * CS, CS)])
            return carry

        lax.fori_loop(0, 2 * J, body, 0)

    return k(ys, pos3d, w0e, w1e)


def _schedule(counts):
    ends = jnp.cumsum(counts)
    starts = ends - counts
    nonempty = counts > 0
    ft = starts // TM
    lt = jnp.maximum(ends - 1, 0) // TM
    nv = jnp.where(nonempty, lt - ft + 1, 0)
    vend = jnp.cumsum(nv)
    vstart = vend - nv
    total = vend[-1]
    i = jnp.arange(V, dtype=jnp.int32)
    g = jnp.searchsorted(vend, i, side="right").astype(jnp.int32)
    gc = jnp.clip(g, 0, E - 1)
    j = i - vstart[gc]
    tile = ft[gc] + j
    lo = jnp.maximum(starts[gc], tile * TM)
    hi = jnp.minimum(ends[gc], (tile + 1) * TM)
    real = i < total
    exp_last = jnp.take(gc, jnp.maximum(total - 1, 0))
    expert = jnp.where(real, gc, exp_last)
    tile = jnp.where(real, tile, NT - 1)
    lo = jnp.where(real, lo, 0)
    hi = jnp.where(real, hi, 0)
    fv = jnp.concatenate(
        [jnp.ones((1,), jnp.int32),
         (tile[1:] != tile[:-1]).astype(jnp.int32)])
    return (tile.astype(jnp.int32), expert.astype(jnp.int32),
            lo.astype(jnp.int32), hi.astype(jnp.int32), fv)


def kernel(x, W_router, W_experts, b_experts):
    wr_pad = jnp.zeros((D, 128), jnp.float32).at[:, :E].set(W_router)
    meta, cnt = _router(x, wr_pad)
    e0 = meta[:, 0].astype(jnp.int32)
    e1 = meta[:, 1].astype(jnp.int32)
    w0 = meta[:, 2]
    w1 = meta[:, 3]
    rank0 = meta[:, 4].astype(jnp.int32)
    rank1 = meta[:, 5].astype(jnp.int32)
    counts = cnt[NB - 1, 0, :E].astype(jnp.int32)

    gs = jnp.cumsum(counts) - counts
    pos0 = jnp.take(gs, e0) + rank0
    pos1 = jnp.take(gs, e1) + rank1
    pos3d = jnp.concatenate(
        [pos0.reshape(NW, J, RSUB), pos1.reshape(NW, J, RSUB)], axis=1)
    w0e = meta[:, 64:80]
    w1e = meta[:, 80:96]
    vt, ve, vlo, vhi, vfv = _schedule(counts)

    xs = _dispatch(x, pos3d)

    b3 = b_experts.reshape(E, 1, D)
    ys = _gmm(vt, ve, vlo, vhi, vfv, xs, W_experts, b3)

    return _combine(ys, pos3d, w0e, w1e)


# speedup vs baseline: 1.7163x; 1.2743x over previous
import functools

import jax
import jax.numpy as jnp
from jax import lax
from jax.experimental import pallas as pl
from jax.experimental.pallas import tpu as pltpu
from jax.experimental.pallas import tpu_sc as plsc

E = 8
K = 2
D = 2048
N = 8192
M = N * K

RBLK = 512
NB = N // RBLK

TM = 512
NT = M // TM
V = NT + E - 1

NC = 2
NS = 16
NW = NC * NS
TPW = N // NW
RSUB = 32
J = TPW // RSUB
CS = 16


def _router_body(x_ref, wr_ref, meta_ref, cnt_ref, carry_ref):
    b = pl.program_id(0)

    @pl.when(b == 0)
    def _():
        carry_ref[...] = jnp.zeros_like(carry_ref)

    xb = x_ref[...]
    logits = jnp.dot(xb, wr_ref[...], preferred_element_type=jnp.float32)
    lane = lax.broadcasted_iota(jnp.int32, (RBLK, 128), 1)
    valid = lane < E
    neg = jnp.float32(-1e30)
    lm = jnp.where(valid, logits, neg)
    m0 = jnp.max(lm, axis=1, keepdims=True)
    is0 = (lm == m0) & valid
    e0 = jnp.min(jnp.where(is0, lane, 127), axis=1, keepdims=True)
    sel0 = lane == e0
    lm1 = jnp.where(sel0, neg, lm)
    m1 = jnp.max(lm1, axis=1, keepdims=True)
    is1 = (lm1 == m1) & valid
    e1 = jnp.min(jnp.where(is1, lane, 127), axis=1, keepdims=True)
    sel1 = lane == e1
    t = jnp.exp(m1 - m0)
    w0 = 1.0 / (1.0 + t)
    w1 = t / (1.0 + t)
    oh = sel0.astype(jnp.float32) + sel1.astype(jnp.float32)
    ri = lax.broadcasted_iota(jnp.int32, (RBLK, RBLK), 0)
    ci = lax.broadcasted_iota(jnp.int32, (RBLK, RBLK), 1)
    tri = (ri > ci).astype(jnp.float32)
    excl = jnp.dot(tri, oh, preferred_element_type=jnp.float32)
    carry = carry_ref[0:1, :]
    rank = excl + carry
    rank0 = jnp.sum(jnp.where(sel0, rank, 0.0), axis=1, keepdims=True)
    rank1 = jnp.sum(jnp.where(sel1, rank, 0.0), axis=1, keepdims=True)
    tot = carry + jnp.sum(oh, axis=0, keepdims=True)
    carry_ref[...] = jnp.broadcast_to(tot, (8, 128))
    cnt_ref[...] = jnp.broadcast_to(tot, (8, 128)).reshape(1, 8, 128)
    meta_ref[...] = (
        jnp.where(lane == 0, e0.astype(jnp.float32), 0.0)
        + jnp.where(lane == 1, e1.astype(jnp.float32), 0.0)
        + jnp.where(lane == 2, w0, 0.0)
        + jnp.where(lane == 3, w1, 0.0)
        + jnp.where(lane == 4, rank0, 0.0)
        + jnp.where(lane == 5, rank1, 0.0)
        + jnp.where((lane >= 64) & (lane < 80), w0, 0.0)
        + jnp.where((lane >= 80) & (lane < 96), w1, 0.0)
    )


def _router(x, wr_pad, interpret=False):
    return pl.pallas_call(
        _router_body,
        grid=(NB,),
        in_specs=[
            pl.BlockSpec((RBLK, D), lambda b: (b, 0)),
            pl.BlockSpec((D, 128), lambda b: (0, 0)),
        ],
        out_specs=[
            pl.BlockSpec((RBLK, 128), lambda b: (b, 0)),
            pl.BlockSpec((1, 8, 128), lambda b: (b, 0, 0)),
        ],
        out_shape=[
            jax.ShapeDtypeStruct((N, 128), jnp.float32),
            jax.ShapeDtypeStruct((NB, 8, 128), jnp.float32),
        ],
        scratch_shapes=[pltpu.VMEM((8, 128), jnp.float32)],
        interpret=interpret,
    )(x, wr_pad)


def _gmm_body(vt_ref, ve_ref, vlo_ref, vhi_ref, vfv_ref, xs_ref, w_ref, b_ref,
              out_ref):
    v = pl.program_id(0)
    tile = vt_ref[v]
    lo = vlo_ref[v]
    hi = vhi_ref[v]
    fv = vfv_ref[v]

    @pl.when(hi > lo)
    def _():
        rows = lax.broadcasted_iota(jnp.int32, (TM, 1), 0) + tile * TM
        m = (rows >= lo) & (rows < hi)
        acc = jnp.dot(xs_ref[...], w_ref[0], preferred_element_type=jnp.float32)
        acc = jnp.where(m, acc + b_ref[0], 0.0)
        prev = out_ref[...]
        out_ref[...] = jnp.where(fv == 1, acc, prev + acc)


def _gmm(vt, ve, vlo, vhi, vfv, xs, w_experts, b3, interpret=False):
    spec = pltpu.PrefetchScalarGridSpec(
        num_scalar_prefetch=5,
        grid=(V,),
        in_specs=[
            pl.BlockSpec((TM, D), lambda v, vt, ve, vlo, vhi, vfv: (vt[v], 0)),
            pl.BlockSpec((1, D, D),
                         lambda v, vt, ve, vlo, vhi, vfv: (ve[v], 0, 0)),
            pl.BlockSpec((1, 1, D),
                         lambda v, vt, ve, vlo, vhi, vfv: (ve[v], 0, 0)),
        ],
        out_specs=pl.BlockSpec((TM, D),
                               lambda v, vt, ve, vlo, vhi, vfv: (vt[v], 0)),
    )
    return pl.pallas_call(
        _gmm_body,
        grid_spec=spec,
        out_shape=jax.ShapeDtypeStruct((M, D), jnp.float32),
        compiler_params=pltpu.CompilerParams(
            vmem_limit_bytes=100 * 1024 * 1024),
        interpret=interpret,
    )(vt, ve, vlo, vhi, vfv, xs, w_experts, b3)


def _dispatch(x, pos3d):
    mesh = plsc.VectorSubcoreMesh(core_axis_name="c", subcore_axis_name="s",
                                  num_cores=NC, num_subcores=NS)

    @functools.partial(
        pl.kernel,
        out_type=jax.ShapeDtypeStruct((M, D), jnp.float32),
        mesh=mesh,
        scratch_types=[
            pltpu.VMEM((2 * J, RSUB), jnp.int32),
            pltpu.VMEM((RSUB, D), jnp.float32),
            pltpu.SemaphoreType.DMA,
            pltpu.SemaphoreType.DMA,
        ],
    )
    def k(x_hbm, pos_hbm, xs_hbm, idx_v, rows_v, s0, s1):
        wid = lax.axis_index("s") * NC + lax.axis_index("c")
        base = wid * TPW
        pltpu.sync_copy(pos_hbm.at[wid], idx_v)

        def body(j, carry):
            pltpu.sync_copy(x_hbm.at[pl.ds(base + j * RSUB, RSUB)], rows_v)
            c0 = pltpu.async_copy(rows_v, xs_hbm.at[idx_v.at[j]], s0)
            c1 = pltpu.async_copy(rows_v, xs_hbm.at[idx_v.at[J + j]], s1)
            c0.wait()
            c1.wait()
            return carry

        lax.fori_loop(0, J, body, 0)

    return k(x, pos3d)


def _gather_pair(ys, pos3d):
    mesh = plsc.VectorSubcoreMesh(core_axis_name="c", subcore_axis_name="s",
                                  num_cores=NC, num_subcores=NS)

    @functools.partial(
        pl.kernel,
        out_type=[
            jax.ShapeDtypeStruct((N, D), jnp.float32),
            jax.ShapeDtypeStruct((N, D), jnp.float32),
        ],
        mesh=mesh,
        scratch_types=[
            pltpu.VMEM((2 * J, RSUB), jnp.int32),
            pltpu.VMEM((CS, D), jnp.float32),
            pltpu.VMEM((CS, D), jnp.float32),
            pltpu.SemaphoreType.DMA,
            pltpu.SemaphoreType.DMA,
        ],
    )
    def k(ys_hbm, pos_hbm, g0_hbm, g1_hbm, idx_v, r0_v, r1_v, s0, s1):
        wid = lax.axis_index("s") * NC + lax.axis_index("c")
        base = wid * TPW
        pltpu.sync_copy(pos_hbm.at[wid], idx_v)

        def body(h, carry):
            j = h // 2
            half = (h % 2) * CS
            tok = base + j * RSUB + half
            c0 = pltpu.async_copy(ys_hbm.at[idx_v.at[j, pl.ds(half, CS)]],
                                  r0_v, s0)
            c1 = pltpu.async_copy(ys_hbm.at[idx_v.at[J + j, pl.ds(half, CS)]],
                                  r1_v, s1)
            c0.wait()
            pltpu.sync_copy(r0_v, g0_hbm.at[pl.ds(tok, CS)])
            c1.wait()
            pltpu.sync_copy(r1_v, g1_hbm.at[pl.ds(tok, CS)])
            return carry

        lax.fori_loop(0, 2 * J, body, 0)

    return k(ys, pos3d)


def _wadd_body(m_ref, g0_ref, g1_ref, o_ref):
    w0 = m_ref[:, 2:3]
    w1 = m_ref[:, 3:4]
    o_ref[...] = w0 * g0_ref[...] + w1 * g1_ref[...]


def _wadd(meta, g0, g1, interpret=False):
    return pl.pallas_call(
        _wadd_body,
        grid=(NB,),
        in_specs=[
            pl.BlockSpec((RBLK, 128), lambda b: (b, 0)),
            pl.BlockSpec((RBLK, D), lambda b: (b, 0)),
            pl.BlockSpec((RBLK, D), lambda b: (b, 0)),
        ],
        out_specs=pl.BlockSpec((RBLK, D), lambda b: (b, 0)),
        out_shape=jax.ShapeDtypeStruct((N, D), jnp.float32),
        interpret=interpret,
    )(meta, g0, g1)


def _schedule(counts):
    ends = jnp.cumsum(counts)
    starts = ends - counts
    nonempty = counts > 0
    ft = starts // TM
    lt = jnp.maximum(ends - 1, 0) // TM
    nv = jnp.where(nonempty, lt - ft + 1, 0)
    vend = jnp.cumsum(nv)
    vstart = vend - nv
    total = vend[-1]
    i = jnp.arange(V, dtype=jnp.int32)
    g = jnp.searchsorted(vend, i, side="right").astype(jnp.int32)
    gc = jnp.clip(g, 0, E - 1)
    j = i - vstart[gc]
    tile = ft[gc] + j
    lo = jnp.maximum(starts[gc], tile * TM)
    hi = jnp.minimum(ends[gc], (tile + 1) * TM)
    real = i < total
    exp_last = jnp.take(gc, jnp.maximum(total - 1, 0))
    expert = jnp.where(real, gc, exp_last)
    tile = jnp.where(real, tile, NT - 1)
    lo = jnp.where(real, lo, 0)
    hi = jnp.where(real, hi, 0)
    fv = jnp.concatenate(
        [jnp.ones((1,), jnp.int32),
         (tile[1:] != tile[:-1]).astype(jnp.int32)])
    return (tile.astype(jnp.int32), expert.astype(jnp.int32),
            lo.astype(jnp.int32), hi.astype(jnp.int32), fv)


def kernel(x, W_router, W_experts, b_experts):
    wr_pad = jnp.zeros((D, 128), jnp.float32).at[:, :E].set(W_router)
    meta, cnt = _router(x, wr_pad)
    e0 = meta[:, 0].astype(jnp.int32)
    e1 = meta[:, 1].astype(jnp.int32)
    w0 = meta[:, 2]
    w1 = meta[:, 3]
    rank0 = meta[:, 4].astype(jnp.int32)
    rank1 = meta[:, 5].astype(jnp.int32)
    counts = cnt[NB - 1, 0, :E].astype(jnp.int32)

    gs = jnp.cumsum(counts) - counts
    pos0 = jnp.take(gs, e0) + rank0
    pos1 = jnp.take(gs, e1) + rank1
    pos3d = jnp.concatenate(
        [pos0.reshape(NW, J, RSUB), pos1.reshape(NW, J, RSUB)], axis=1)
    vt, ve, vlo, vhi, vfv = _schedule(counts)

    xs = _dispatch(x, pos3d)

    b3 = b_experts.reshape(E, 1, D)
    ys = _gmm(vt, ve, vlo, vhi, vfv, xs, W_experts, b3)

    g0, g1 = _gather_pair(ys, pos3d)
    return _wadd(meta, g0, g1)
